# baseline (speedup 1.0000x reference)
import functools

import jax
import jax.numpy as jnp
import numpy as np
from jax import lax
from jax.experimental import pallas as pl
from jax.experimental.pallas import tpu as pltpu
from jax.experimental.pallas import tpu_sc as plsc

N = 100000
D = 128
DL = 16
S = 16
B = 8192
EMB = 128
H = 4
L = 2
FF = 256
C = 40
T = S + 1
BT = B * T
DH = EMB // H

BB = 128
R = BB * T
GS = 8
RG = GS * T
NG = BB // GS
GS2 = 16
RG2 = GS2 * T
NG2 = BB // GS2
NBLK = B // BB

_SC_MESH = functools.partial(
    plsc.VectorSubcoreMesh, core_axis_name="c", subcore_axis_name="s"
)


def _sc_gather_nbrows(neigh_p, rows2d):
    W = 256

    @functools.partial(
        pl.kernel,
        out_type=jax.ShapeDtypeStruct((B, 128), jnp.int32),
        mesh=_SC_MESH(),
    )
    def k(tab_hbm, i_hbm, o_hbm):
        def body(i_vmem, o_vmem):
            pltpu.sync_copy(tab_hbm.at[i_vmem.at[0]], o_vmem)

        pltpu.emit_pipeline(
            body,
            grid=(B // W,),
            in_specs=[pl.BlockSpec((1, W), lambda i: (0, i))],
            out_specs=[pl.BlockSpec((W, 128), lambda i: (i, 0))],
            core_axis_name=("c", "s"),
            dimension_semantics=(pltpu.PARALLEL,),
        )(i_hbm, o_hbm)

    return k(neigh_p, rows2d)


def _sc_gather_xin(xin, tok2d):
    W = 256

    @functools.partial(
        pl.kernel,
        out_type=jax.ShapeDtypeStruct((BT, EMB), jnp.float32),
        mesh=_SC_MESH(),
    )
    def k(tab_hbm, i_hbm, o_hbm):
        def body(i_vmem, o_vmem):
            pltpu.sync_copy(tab_hbm.at[i_vmem.at[0]], o_vmem)

        pltpu.emit_pipeline(
            body,
            grid=(BT // W,),
            in_specs=[pl.BlockSpec((1, W), lambda i: (0, i))],
            out_specs=[pl.BlockSpec((W, EMB), lambda i: (i, 0))],
            core_axis_name=("c", "s"),
            dimension_semantics=(pltpu.PARALLEL,),
        )(i_hbm, o_hbm)

    return k(xin, tok2d)


def _proj_body(feat_ref, lap_ref, wif_ref, wil_ref, o_ref):
    o_ref[...] = (
        jnp.dot(feat_ref[...], wif_ref[...],
                preferred_element_type=jnp.float32)
        + jnp.dot(lap_ref[...], wil_ref[...],
                  preferred_element_type=jnp.float32))


def _proj_kernel(feat, lap, w_in):
    blk = 8192
    nb = (N + 1 + blk - 1) // blk
    return pl.pallas_call(
        _proj_body,
        grid=(nb,),
        in_specs=[
            pl.BlockSpec((blk, D), lambda i: (i, 0)),
            pl.BlockSpec((blk, DL), lambda i: (i, 0)),
            pl.BlockSpec((D, EMB), lambda i: (0, 0)),
            pl.BlockSpec((DL, EMB), lambda i: (0, 0)),
        ],
        out_specs=pl.BlockSpec((blk, EMB), lambda i: (i, 0)),
        out_shape=jax.ShapeDtypeStruct((N + 1, EMB), jnp.float32),
        compiler_params=pltpu.CompilerParams(
            dimension_semantics=("parallel",)),
    )(feat, lap, w_in[:D], w_in[D:])


def _ln(z):
    m = jnp.mean(z, axis=-1, keepdims=True)
    v = jnp.mean((z - m) * (z - m), axis=-1, keepdims=True)
    return (z - m) / jnp.sqrt(v + 1e-5)


def _softmax(s):
    m = jnp.max(s, axis=-1, keepdims=True)
    e = jnp.exp(s - m)
    return e / jnp.sum(e, axis=-1, keepdims=True)


def _tc_body(px_ref, wq_ref, wk_ref, wv_ref,
             wo_ref, w1_ref, w2_ref, wd_ref, bd_ref, out_ref,
             q_ref, k_ref, v_ref, o_ref, qs_ref, o2_ref):
    f32 = jnp.float32
    bf16 = jnp.bfloat16
    scale = f32(1.0 / np.sqrt(DH))

    x = px_ref[...]

    r1 = lax.broadcasted_iota(jnp.int32, (RG, RG), 0)
    c1 = lax.broadcasted_iota(jnp.int32, (RG, RG), 1)
    mask1 = jnp.where((r1 // T) == (c1 // T), f32(0.0), f32(-1e30))
    r2 = lax.broadcasted_iota(jnp.int32, (GS2, RG2), 0)
    c2 = lax.broadcasted_iota(jnp.int32, (GS2, RG2), 1)
    mask2 = jnp.where((c2 // T) == r2, f32(0.0), f32(-1e30))

    xb = x.astype(bf16)
    q_ref[...] = jnp.dot(xb, wq_ref[0],
                         preferred_element_type=jnp.float32).astype(bf16)
    k_ref[...] = jnp.dot(xb, wk_ref[0],
                         preferred_element_type=jnp.float32).astype(bf16)
    v_ref[...] = jnp.dot(xb, wv_ref[0],
                         preferred_element_type=jnp.float32).astype(bf16)

    def grp0(g, carry):
        base = g * RG
        for h in range(H):
            cs = slice(h * DH, (h + 1) * DH)
            qh = q_ref[pl.ds(base, RG), cs]
            kh = k_ref[pl.ds(base, RG), cs]
            vh = v_ref[pl.ds(base, RG), cs]
            s = lax.dot_general(qh, kh, (((1,), (1,)), ((), ())),
                                preferred_element_type=f32) * scale + mask1
            p = _softmax(s).astype(bf16)
            o_ref[pl.ds(base, RG), cs] = jnp.dot(
                p, vh, preferred_element_type=f32)
        return carry

    lax.fori_loop(0, NG, grp0, 0)

    x = _ln(x + jnp.dot(o_ref[...].astype(bf16), wo_ref[0],
                        preferred_element_type=f32))
    xb = x.astype(bf16)
    ff = jnp.dot(jax.nn.relu(jnp.dot(xb, w1_ref[0],
                                     preferred_element_type=f32)).astype(bf16),
                 w2_ref[0], preferred_element_type=f32)
    x = _ln(x + ff)

    rs = lax.broadcasted_iota(jnp.int32, (BB, R), 0)
    cc = lax.broadcasted_iota(jnp.int32, (BB, R), 1)
    sel = jnp.where(cc == rs * T, f32(1.0), f32(0.0)).astype(bf16)
    xb = x.astype(bf16)
    xs = jnp.dot(sel, xb, preferred_element_type=f32)

    qs_ref[...] = jnp.dot(xs.astype(bf16), wq_ref[1],
                          preferred_element_type=jnp.float32).astype(bf16)
    k_ref[...] = jnp.dot(xb, wk_ref[1],
                         preferred_element_type=jnp.float32).astype(bf16)
    v_ref[...] = jnp.dot(xb, wv_ref[1],
                         preferred_element_type=jnp.float32).astype(bf16)

    def grp1(g, carry):
        base = g * RG2
        sbase = g * GS2
        for h in range(H):
            cs = slice(h * DH, (h + 1) * DH)
            qh = qs_ref[pl.ds(sbase, GS2), cs]
            kh = k_ref[pl.ds(base, RG2), cs]
            vh = v_ref[pl.ds(base, RG2), cs]
            s = lax.dot_general(qh, kh, (((1,), (1,)), ((), ())),
                                preferred_element_type=f32) * scale + mask2
            p = _softmax(s).astype(bf16)
            o2_ref[pl.ds(sbase, GS2), cs] = jnp.dot(
                p, vh, preferred_element_type=f32)
        return carry

    lax.fori_loop(0, NG2, grp1, 0)

    xs = _ln(xs + jnp.dot(o2_ref[...].astype(bf16), wo_ref[1],
                          preferred_element_type=f32))
    xsb = xs.astype(bf16)
    ff2 = jnp.dot(jax.nn.relu(jnp.dot(xsb, w1_ref[1],
                                      preferred_element_type=f32)
                              ).astype(bf16),
                  w2_ref[1], preferred_element_type=f32)
    xs = _ln(xs + ff2)

    out_ref[...] = (jnp.dot(xs.astype(bf16), wd_ref[...],
                            preferred_element_type=f32)
                    + bd_ref[...])


def _tc_transformer(px, wq, wk, wv, wo, w1, w2, wd, bd):
    f32 = jnp.float32
    bf16 = jnp.bfloat16
    bd2 = bd.reshape(1, C)
    wq, wk, wv, wo, w1, w2, wd = (
        w.astype(bf16) for w in (wq, wk, wv, wo, w1, w2, wd))

    return pl.pallas_call(
        _tc_body,
        grid=(NBLK,),
        in_specs=[
            pl.BlockSpec((R, EMB), lambda i: (i, 0)),
            pl.BlockSpec((L, EMB, EMB), lambda i: (0, 0, 0)),
            pl.BlockSpec((L, EMB, EMB), lambda i: (0, 0, 0)),
            pl.BlockSpec((L, EMB, EMB), lambda i: (0, 0, 0)),
            pl.BlockSpec((L, EMB, EMB), lambda i: (0, 0, 0)),
            pl.BlockSpec((L, EMB, FF), lambda i: (0, 0, 0)),
            pl.BlockSpec((L, FF, EMB), lambda i: (0, 0, 0)),
            pl.BlockSpec((EMB, C), lambda i: (0, 0)),
            pl.BlockSpec((1, C), lambda i: (0, 0)),
        ],
        out_specs=pl.BlockSpec((BB, C), lambda i: (i, 0)),
        out_shape=jax.ShapeDtypeStruct((B, C), f32),
        scratch_shapes=[
            pltpu.VMEM((R, EMB), bf16),
            pltpu.VMEM((R, EMB), bf16),
            pltpu.VMEM((R, EMB), bf16),
            pltpu.VMEM((R, EMB), f32),
            pltpu.VMEM((BB, EMB), bf16),
            pltpu.VMEM((BB, EMB), f32),
        ],
        compiler_params=pltpu.CompilerParams(
            dimension_semantics=("parallel",)),
    )(px, wq, wk, wv, wo, w1, w2, wd, bd2)


def kernel(nodes, neigh, feat, lap, W_in, Wq, Wk, Wv, Wo, W1, W2,
           W_dense, b_dense):
    nodes32 = nodes.astype(jnp.int32)
    neigh32 = neigh.astype(jnp.int32)

    neigh_p = neigh32.reshape(N // 8, 8 * S)
    nbrows = _sc_gather_nbrows(neigh_p, (nodes32 // 8).reshape(1, B))
    j = nodes32[:, None] % 8
    nb = nbrows[:, 0:S]
    for jj in range(1, 8):
        nb = jnp.where(j == jj, nbrows[:, jj * S:(jj + 1) * S], nb)
    tok = jnp.concatenate([nodes32[:, None], nb], axis=1)
    tok2d = tok.reshape(1, BT)

    xin = _proj_kernel(feat, lap, W_in)
    px = _sc_gather_xin(xin, tok2d)

    return _tc_transformer(px, Wq, Wk, Wv, Wo, W1, W2, W_dense, b_dense)


# speedup vs baseline: 2.7943x; 2.7943x over previous
import functools

import jax
import jax.numpy as jnp
import numpy as np
from jax import lax
from jax.experimental import pallas as pl
from jax.experimental.pallas import tpu as pltpu
from jax.experimental.pallas import tpu_sc as plsc

N = 100000
D = 128
DL = 16
S = 16
B = 8192
EMB = 128
H = 4
L = 2
FF = 256
C = 40
T = S + 1
BT = B * T
DH = EMB // H

BB = 128
R = BB * T
GS = 8
RG = GS * T
NG = BB // GS
GS2 = 8
RG2 = GS2 * T
NG2 = BB // GS2
NBLK = B // BB

_SC_MESH = functools.partial(
    plsc.VectorSubcoreMesh, core_axis_name="c", subcore_axis_name="s"
)


def _sc_gather_nbrows(neigh_p, rows2d):
    W = 256

    @functools.partial(
        pl.kernel,
        out_type=jax.ShapeDtypeStruct((B, 128), jnp.int32),
        mesh=_SC_MESH(),
    )
    def k(tab_hbm, i_hbm, o_hbm):
        def body(i_vmem, o_vmem):
            pltpu.sync_copy(tab_hbm.at[i_vmem.at[0]], o_vmem)

        pltpu.emit_pipeline(
            body,
            grid=(B // W,),
            in_specs=[pl.BlockSpec((1, W), lambda i: (0, i))],
            out_specs=[pl.BlockSpec((W, 128), lambda i: (i, 0))],
            core_axis_name=("c", "s"),
            dimension_semantics=(pltpu.PARALLEL,),
        )(i_hbm, o_hbm)

    return k(neigh_p, rows2d)


def _sc_gather_xin(xin, tok2d):
    W = 256

    @functools.partial(
        pl.kernel,
        out_type=jax.ShapeDtypeStruct((BT, EMB), jnp.float32),
        mesh=_SC_MESH(),
    )
    def k(tab_hbm, i_hbm, o_hbm):
        def body(i_vmem, o_vmem):
            pltpu.sync_copy(tab_hbm.at[i_vmem.at[0]], o_vmem)

        pltpu.emit_pipeline(
            body,
            grid=(BT // W,),
            in_specs=[pl.BlockSpec((1, W), lambda i: (0, i))],
            out_specs=[pl.BlockSpec((W, EMB), lambda i: (i, 0))],
            core_axis_name=("c", "s"),
            dimension_semantics=(pltpu.PARALLEL,),
        )(i_hbm, o_hbm)

    return k(xin, tok2d)


def _proj_body(feat_ref, lap_ref, wif_ref, wil_ref, o_ref):
    o_ref[...] = (
        jnp.dot(feat_ref[...], wif_ref[...],
                preferred_element_type=jnp.float32)
        + jnp.dot(lap_ref[...], wil_ref[...],
                  preferred_element_type=jnp.float32))


def _proj_kernel(feat, lap, w_in):
    blk = 8192
    nb = (N + 1 + blk - 1) // blk
    return pl.pallas_call(
        _proj_body,
        grid=(nb,),
        in_specs=[
            pl.BlockSpec((blk, D), lambda i: (i, 0)),
            pl.BlockSpec((blk, DL), lambda i: (i, 0)),
            pl.BlockSpec((D, EMB), lambda i: (0, 0)),
            pl.BlockSpec((DL, EMB), lambda i: (0, 0)),
        ],
        out_specs=pl.BlockSpec((blk, EMB), lambda i: (i, 0)),
        out_shape=jax.ShapeDtypeStruct((N + 1, EMB), jnp.float32),
        compiler_params=pltpu.CompilerParams(
            dimension_semantics=("parallel",)),
    )(feat, lap, w_in[:D], w_in[D:])


def _ln(z):
    m = jnp.mean(z, axis=-1, keepdims=True)
    v = jnp.mean(z * z, axis=-1, keepdims=True) - m * m
    return (z - m) / jnp.sqrt(v + 1e-5)


def _softmax(s):
    e = jnp.exp(s)
    return e * (1.0 / jnp.sum(e, axis=-1, keepdims=True))


def _attn_group(q_grp, k_grp, v_grp, hm, ms, hmo, nrow):
    f32 = jnp.float32
    bf16 = jnp.bfloat16
    qst = jnp.concatenate([q_grp] * H, axis=0) * hm
    s = lax.dot_general(qst, k_grp, (((1,), (1,)), ((), ())),
                        preferred_element_type=f32) + ms
    p = _softmax(s).astype(bf16)
    av = jnp.dot(p, v_grp, preferred_element_type=f32)
    o = av[0:nrow] * hmo[0:1]
    for h in range(1, H):
        o = o + av[h * nrow:(h + 1) * nrow] * hmo[h:h + 1]
    return o


def _tc_body(px_ref, ms1_ref, ms2_ref, sel_ref, hm1_ref, hm2_ref, hmo_ref,
             wq_ref, wkv_ref, wo_ref, w1_ref, w2_ref, wd_ref, bd_ref,
             out_ref):
    f32 = jnp.float32
    bf16 = jnp.bfloat16

    x = px_ref[...]
    ms1 = ms1_ref[...]
    ms2 = ms2_ref[...]
    hm1 = hm1_ref[...]
    hm2 = hm2_ref[...]
    hmo = hmo_ref[...]

    xb = x.astype(bf16)
    q = jnp.dot(xb, wq_ref[0], preferred_element_type=f32).astype(bf16)
    kv = jnp.dot(xb, wkv_ref[0], preferred_element_type=f32).astype(bf16)

    oparts = []
    for g in range(NG):
        base = g * RG
        q_grp = lax.slice(q, (base, 0), (base + RG, EMB))
        k_grp = lax.slice(kv, (base, 0), (base + RG, EMB))
        v_grp = lax.slice(kv, (base, EMB), (base + RG, 2 * EMB))
        oparts.append(_attn_group(q_grp, k_grp, v_grp, hm1, ms1, hmo, RG))
    o = jnp.concatenate(oparts, axis=0)

    x = _ln(x + jnp.dot(o.astype(bf16), wo_ref[0],
                        preferred_element_type=f32))
    xb = x.astype(bf16)
    ff = jnp.dot(jax.nn.relu(jnp.dot(xb, w1_ref[0],
                                     preferred_element_type=f32)).astype(bf16),
                 w2_ref[0], preferred_element_type=f32)
    x = _ln(x + ff)

    xb = x.astype(bf16)
    xs = jnp.dot(sel_ref[...], xb, preferred_element_type=f32)

    q2 = jnp.dot(xs.astype(bf16), wq_ref[1],
                 preferred_element_type=f32).astype(bf16)
    kv2 = jnp.dot(xb, wkv_ref[1], preferred_element_type=f32).astype(bf16)

    o2parts = []
    for g in range(NG2):
        base = g * RG2
        sbase = g * GS2
        q_grp = lax.slice(q2, (sbase, 0), (sbase + GS2, EMB))
        k_grp = lax.slice(kv2, (base, 0), (base + RG2, EMB))
        v_grp = lax.slice(kv2, (base, EMB), (base + RG2, 2 * EMB))
        o2parts.append(_attn_group(q_grp, k_grp, v_grp, hm2, ms2, hmo, GS2))
    o2 = jnp.concatenate(o2parts, axis=0)

    xs = _ln(xs + jnp.dot(o2.astype(bf16), wo_ref[1],
                          preferred_element_type=f32))
    xsb = xs.astype(bf16)
    ff2 = jnp.dot(jax.nn.relu(jnp.dot(xsb, w1_ref[1],
                                      preferred_element_type=f32)
                              ).astype(bf16),
                  w2_ref[1], preferred_element_type=f32)
    xs = _ln(xs + ff2)

    out_ref[...] = (jnp.dot(xs.astype(bf16), wd_ref[...],
                            preferred_element_type=f32)
                    + bd_ref[...])


def _tc_transformer(px, wq, wk, wv, wo, w1, w2, wd, bd):
    f32 = jnp.float32
    bf16 = jnp.bfloat16
    bd2 = bd.reshape(1, C)
    wkv = jnp.concatenate([wk, wv], axis=2)
    wq, wkv, wo, w1, w2, wd = (
        w.astype(bf16) for w in (wq, wkv, wo, w1, w2, wd))

    scale = 1.0 / np.sqrt(DH)

    def band_mask(nrow):
        r = lax.broadcasted_iota(jnp.int32, (H * nrow, RG), 0) % nrow
        c = lax.broadcasted_iota(jnp.int32, (H * nrow, RG), 1)
        if nrow == RG:
            ok = (r // T) == (c // T)
        else:
            ok = (c // T) == r
        return jnp.where(ok, 0.0, -1e30).astype(f32)

    def head_mask(nrow, val):
        hr = lax.broadcasted_iota(jnp.int32, (H * nrow, EMB), 0) // nrow
        lane = lax.broadcasted_iota(jnp.int32, (H * nrow, EMB), 1) // DH
        return jnp.where(hr == lane, val, 0.0).astype(f32)

    ms1 = band_mask(RG)
    ms2 = band_mask(GS2)
    hm1 = head_mask(RG, scale).astype(bf16)
    hm2 = head_mask(GS2, scale).astype(bf16)
    hro = lax.broadcasted_iota(jnp.int32, (H, EMB), 0)
    lno = lax.broadcasted_iota(jnp.int32, (H, EMB), 1) // DH
    hmo = jnp.where(hro == lno, 1.0, 0.0).astype(f32)
    rs = lax.broadcasted_iota(jnp.int32, (BB, R), 0)
    cc = lax.broadcasted_iota(jnp.int32, (BB, R), 1)
    sel = jnp.where(cc == rs * T, 1.0, 0.0).astype(bf16)

    return pl.pallas_call(
        _tc_body,
        grid=(NBLK,),
        in_specs=[
            pl.BlockSpec((R, EMB), lambda i: (i, 0)),
            pl.BlockSpec((H * RG, RG), lambda i: (0, 0)),
            pl.BlockSpec((H * GS2, RG), lambda i: (0, 0)),
            pl.BlockSpec((BB, R), lambda i: (0, 0)),
            pl.BlockSpec((H * RG, EMB), lambda i: (0, 0)),
            pl.BlockSpec((H * GS2, EMB), lambda i: (0, 0)),
            pl.BlockSpec((H, EMB), lambda i: (0, 0)),
            pl.BlockSpec((L, EMB, EMB), lambda i: (0, 0, 0)),
            pl.BlockSpec((L, EMB, 2 * EMB), lambda i: (0, 0, 0)),
            pl.BlockSpec((L, EMB, EMB), lambda i: (0, 0, 0)),
            pl.BlockSpec((L, EMB, FF), lambda i: (0, 0, 0)),
            pl.BlockSpec((L, FF, EMB), lambda i: (0, 0, 0)),
            pl.BlockSpec((EMB, C), lambda i: (0, 0)),
            pl.BlockSpec((1, C), lambda i: (0, 0)),
        ],
        out_specs=pl.BlockSpec((BB, C), lambda i: (i, 0)),
        out_shape=jax.ShapeDtypeStruct((B, C), f32),
        compiler_params=pltpu.CompilerParams(
            dimension_semantics=("parallel",)),
    )(px, ms1, ms2, sel, hm1, hm2, hmo, wq, wkv, wo, w1, w2, wd, bd2)


def kernel(nodes, neigh, feat, lap, W_in, Wq, Wk, Wv, Wo, W1, W2,
           W_dense, b_dense):
    nodes32 = nodes.astype(jnp.int32)
    neigh32 = neigh.astype(jnp.int32)

    neigh_p = neigh32.reshape(N // 8, 8 * S)
    nbrows = _sc_gather_nbrows(neigh_p, (nodes32 // 8).reshape(1, B))
    j = nodes32[:, None] % 8
    nb = nbrows[:, 0:S]
    for jj in range(1, 8):
        nb = jnp.where(j == jj, nbrows[:, jj * S:(jj + 1) * S], nb)
    tok = jnp.concatenate([nodes32[:, None], nb], axis=1)
    tok2d = tok.reshape(1, BT)

    xin = _proj_kernel(feat, lap, W_in)
    px = _sc_gather_xin(xin, tok2d)

    return _tc_transformer(px, Wq, Wk, Wv, Wo, W1, W2, W_dense, b_dense)


# speedup vs baseline: 2.8829x; 1.0317x over previous
import functools

import jax
import jax.numpy as jnp
import numpy as np
from jax import lax
from jax.experimental import pallas as pl
from jax.experimental.pallas import tpu as pltpu
from jax.experimental.pallas import tpu_sc as plsc

N = 100000
D = 128
DL = 16
S = 16
B = 8192
EMB = 128
H = 4
L = 2
FF = 256
C = 40
T = S + 1
BT = B * T
DH = EMB // H

BB = 128
R = BB * T
GS = 8
RG = GS * T
NG = BB // GS
GS2 = 8
RG2 = GS2 * T
NG2 = BB // GS2
NBLK = B // BB

_SC_MESH = functools.partial(
    plsc.VectorSubcoreMesh, core_axis_name="c", subcore_axis_name="s"
)


def _sc_gather_nbrows(neigh_p, rows2d):
    W = 256

    @functools.partial(
        pl.kernel,
        out_type=jax.ShapeDtypeStruct((B, 128), jnp.int32),
        mesh=_SC_MESH(),
    )
    def k(tab_hbm, i_hbm, o_hbm):
        def body(i_vmem, o_vmem):
            pltpu.sync_copy(tab_hbm.at[i_vmem.at[0]], o_vmem)

        pltpu.emit_pipeline(
            body,
            grid=(B // W,),
            in_specs=[pl.BlockSpec((1, W), lambda i: (0, i))],
            out_specs=[pl.BlockSpec((W, 128), lambda i: (i, 0))],
            core_axis_name=("c", "s"),
            dimension_semantics=(pltpu.PARALLEL,),
        )(i_hbm, o_hbm)

    return k(neigh_p, rows2d)


def _sc_gather_xin(xin, tok2d):
    W = 256
    nt = tok2d.shape[1]

    @functools.partial(
        pl.kernel,
        out_type=jax.ShapeDtypeStruct((nt, EMB), jnp.float32),
        mesh=_SC_MESH(),
    )
    def k(tab_hbm, i_hbm, o_hbm):
        def body(i_vmem, o_vmem):
            pltpu.sync_copy(tab_hbm.at[i_vmem.at[0]], o_vmem)

        pltpu.emit_pipeline(
            body,
            grid=(nt // W,),
            in_specs=[pl.BlockSpec((1, W), lambda i: (0, i))],
            out_specs=[pl.BlockSpec((W, EMB), lambda i: (i, 0))],
            core_axis_name=("c", "s"),
            dimension_semantics=(pltpu.PARALLEL,),
        )(i_hbm, o_hbm)

    return k(xin, tok2d)


def _proj_body(feat_ref, lap_ref, wif_ref, wil_ref, o_ref):
    o_ref[...] = (
        jnp.dot(feat_ref[...], wif_ref[...],
                preferred_element_type=jnp.float32)
        + jnp.dot(lap_ref[...], wil_ref[...],
                  preferred_element_type=jnp.float32))


def _proj_kernel(feat, lap, w_in):
    blk = 8192
    nb = (N + 1 + blk - 1) // blk
    return pl.pallas_call(
        _proj_body,
        grid=(nb,),
        in_specs=[
            pl.BlockSpec((blk, D), lambda i: (i, 0)),
            pl.BlockSpec((blk, DL), lambda i: (i, 0)),
            pl.BlockSpec((D, EMB), lambda i: (0, 0)),
            pl.BlockSpec((DL, EMB), lambda i: (0, 0)),
        ],
        out_specs=pl.BlockSpec((blk, EMB), lambda i: (i, 0)),
        out_shape=jax.ShapeDtypeStruct((N + 1, EMB), jnp.float32),
        compiler_params=pltpu.CompilerParams(
            dimension_semantics=("parallel",)),
    )(feat, lap, w_in[:D], w_in[D:])


def _ln(z):
    m = jnp.mean(z, axis=-1, keepdims=True)
    v = jnp.mean(z * z, axis=-1, keepdims=True) - m * m
    return (z - m) / jnp.sqrt(v + 1e-5)


def _softmax(s):
    e = jnp.exp(s)
    return e * (1.0 / jnp.sum(e, axis=-1, keepdims=True))


def _attn_group(q_grp, k_grp, v_grp, hm, ms, hmo, nrow):
    f32 = jnp.float32
    bf16 = jnp.bfloat16
    qst = jnp.concatenate([q_grp] * H, axis=0) * hm
    s = lax.dot_general(qst, k_grp, (((1,), (1,)), ((), ())),
                        preferred_element_type=f32) + ms
    p = _softmax(s).astype(bf16)
    av = jnp.dot(p, v_grp, preferred_element_type=f32)
    o = av[0:nrow] * hmo[0:1]
    for h in range(1, H):
        o = o + av[h * nrow:(h + 1) * nrow] * hmo[h:h + 1]
    return o


def _tc_body(px_ref, ms1_ref, ms2_ref, sel_ref, hm1_ref, hm2_ref, hmo_ref,
             wq_ref, wkv_ref, wo_ref, w1_ref, w2_ref, wd_ref, bd_ref,
             out_ref):
    f32 = jnp.float32
    bf16 = jnp.bfloat16

    x = px_ref[...]
    xb = x.astype(bf16)
    ms1 = ms1_ref[...]
    ms2 = ms2_ref[...]
    hm1 = hm1_ref[...]
    hm2 = hm2_ref[...]
    hmo = hmo_ref[...]

    q = jnp.dot(xb, wq_ref[0], preferred_element_type=f32).astype(bf16)
    kv = jnp.dot(xb, wkv_ref[0], preferred_element_type=f32).astype(bf16)

    oparts = []
    for g in range(NG):
        base = g * RG
        q_grp = lax.slice(q, (base, 0), (base + RG, EMB))
        k_grp = lax.slice(kv, (base, 0), (base + RG, EMB))
        v_grp = lax.slice(kv, (base, EMB), (base + RG, 2 * EMB))
        oparts.append(_attn_group(q_grp, k_grp, v_grp, hm1, ms1, hmo, RG))
    o = jnp.concatenate(oparts, axis=0)

    x = _ln(x + jnp.dot(o.astype(bf16), wo_ref[0],
                        preferred_element_type=f32))
    xb = x.astype(bf16)
    ff = jnp.dot(jax.nn.relu(jnp.dot(xb, w1_ref[0],
                                     preferred_element_type=f32)).astype(bf16),
                 w2_ref[0], preferred_element_type=f32)
    x = _ln(x + ff)

    xb = x.astype(bf16)
    xs = jnp.dot(sel_ref[...], xb, preferred_element_type=f32)

    q2 = jnp.dot(xs.astype(bf16), wq_ref[1],
                 preferred_element_type=f32).astype(bf16)
    kv2 = jnp.dot(xb, wkv_ref[1], preferred_element_type=f32).astype(bf16)

    o2parts = []
    for g in range(NG2):
        base = g * RG2
        sbase = g * GS2
        q_grp = lax.slice(q2, (sbase, 0), (sbase + GS2, EMB))
        k_grp = lax.slice(kv2, (base, 0), (base + RG2, EMB))
        v_grp = lax.slice(kv2, (base, EMB), (base + RG2, 2 * EMB))
        o2parts.append(_attn_group(q_grp, k_grp, v_grp, hm2, ms2, hmo, GS2))
    o2 = jnp.concatenate(o2parts, axis=0)

    xs = _ln(xs + jnp.dot(o2.astype(bf16), wo_ref[1],
                          preferred_element_type=f32))
    xsb = xs.astype(bf16)
    ff2 = jnp.dot(jax.nn.relu(jnp.dot(xsb, w1_ref[1],
                                      preferred_element_type=f32)
                              ).astype(bf16),
                  w2_ref[1], preferred_element_type=f32)
    xs = _ln(xs + ff2)

    out_ref[...] = (jnp.dot(xs.astype(bf16), wd_ref[...],
                            preferred_element_type=f32)
                    + bd_ref[...])


def _tc_transformer(px, wq, wk, wv, wo, w1, w2, wd, bd):
    f32 = jnp.float32
    bf16 = jnp.bfloat16
    bd2 = bd.reshape(1, C)
    wkv = jnp.concatenate([wk, wv], axis=2)
    wq, wkv, wo, w1, w2, wd = (
        w.astype(bf16) for w in (wq, wkv, wo, w1, w2, wd))

    scale = 1.0 / np.sqrt(DH)

    def band_mask(nrow):
        r = lax.broadcasted_iota(jnp.int32, (H * nrow, RG), 0) % nrow
        c = lax.broadcasted_iota(jnp.int32, (H * nrow, RG), 1)
        if nrow == RG:
            ok = (r // T) == (c // T)
        else:
            ok = (c // T) == r
        return jnp.where(ok, 0.0, -1e30).astype(f32)

    def head_mask(nrow, val):
        hr = lax.broadcasted_iota(jnp.int32, (H * nrow, EMB), 0) // nrow
        lane = lax.broadcasted_iota(jnp.int32, (H * nrow, EMB), 1) // DH
        return jnp.where(hr == lane, val, 0.0).astype(f32)

    ms1 = band_mask(RG)
    ms2 = band_mask(GS2)
    hm1 = head_mask(RG, scale).astype(bf16)
    hm2 = head_mask(GS2, scale).astype(bf16)
    hro = lax.broadcasted_iota(jnp.int32, (H, EMB), 0)
    lno = lax.broadcasted_iota(jnp.int32, (H, EMB), 1) // DH
    hmo = jnp.where(hro == lno, 1.0, 0.0).astype(f32)
    rs = lax.broadcasted_iota(jnp.int32, (BB, R), 0)
    cc = lax.broadcasted_iota(jnp.int32, (BB, R), 1)
    sel = jnp.where(cc == rs * T, 1.0, 0.0).astype(bf16)

    nblk = px.shape[0] // R
    return pl.pallas_call(
        _tc_body,
        grid=(nblk,),
        in_specs=[
            pl.BlockSpec((R, EMB), lambda i: (i, 0)),
            pl.BlockSpec((H * RG, RG), lambda i: (0, 0)),
            pl.BlockSpec((H * GS2, RG), lambda i: (0, 0)),
            pl.BlockSpec((BB, R), lambda i: (0, 0)),
            pl.BlockSpec((H * RG, EMB), lambda i: (0, 0)),
            pl.BlockSpec((H * GS2, EMB), lambda i: (0, 0)),
            pl.BlockSpec((H, EMB), lambda i: (0, 0)),
            pl.BlockSpec((L, EMB, EMB), lambda i: (0, 0, 0)),
            pl.BlockSpec((L, EMB, 2 * EMB), lambda i: (0, 0, 0)),
            pl.BlockSpec((L, EMB, EMB), lambda i: (0, 0, 0)),
            pl.BlockSpec((L, EMB, FF), lambda i: (0, 0, 0)),
            pl.BlockSpec((L, FF, EMB), lambda i: (0, 0, 0)),
            pl.BlockSpec((EMB, C), lambda i: (0, 0)),
            pl.BlockSpec((1, C), lambda i: (0, 0)),
        ],
        out_specs=pl.BlockSpec((BB, C), lambda i: (i, 0)),
        out_shape=jax.ShapeDtypeStruct((nblk * BB, C), f32),
        compiler_params=pltpu.CompilerParams(
            dimension_semantics=("parallel",)),
    )(px, ms1, ms2, sel, hm1, hm2, hmo, wq, wkv, wo, w1, w2, wd, bd2)


def kernel(nodes, neigh, feat, lap, W_in, Wq, Wk, Wv, Wo, W1, W2,
           W_dense, b_dense):
    nodes32 = nodes.astype(jnp.int32)
    neigh32 = neigh.astype(jnp.int32)

    neigh_p = neigh32.reshape(N // 8, 8 * S)
    nbrows = _sc_gather_nbrows(neigh_p, (nodes32 // 8).reshape(1, B))
    j = nodes32[:, None] % 8
    nb = nbrows[:, 0:S]
    for jj in range(1, 8):
        nb = jnp.where(j == jj, nbrows[:, jj * S:(jj + 1) * S], nb)
    tok = jnp.concatenate([nodes32[:, None], nb], axis=1)

    xin = _proj_kernel(feat, lap, W_in)

    NCHUNK = 4
    BC = B // NCHUNK
    outs = []
    for c in range(NCHUNK):
        tok_c = lax.slice(tok, (c * BC, 0), ((c + 1) * BC, T))
        px_c = _sc_gather_xin(xin, tok_c.reshape(1, BC * T))
        outs.append(_tc_transformer(px_c, Wq, Wk, Wv, Wo, W1, W2,
                                    W_dense, b_dense))
    return jnp.concatenate(outs, axis=0)


# speedup vs baseline: 3.3057x; 1.1467x over previous
import functools

import jax
import jax.numpy as jnp
import numpy as np
from jax import lax
from jax.experimental import pallas as pl
from jax.experimental.pallas import tpu as pltpu
from jax.experimental.pallas import tpu_sc as plsc

N = 100000
D = 128
DL = 16
S = 16
B = 8192
EMB = 128
H = 4
L = 2
FF = 256
C = 40
T = S + 1
BT = B * T
DH = EMB // H

BB = 128
R = BB * T
GS = 8
RG = GS * T
NG = BB // GS
GS2 = 8
RG2 = GS2 * T
NG2 = BB // GS2
NBLK = B // BB

_SC_MESH = functools.partial(
    plsc.VectorSubcoreMesh, core_axis_name="c", subcore_axis_name="s"
)


def _sc_gather_nbrows(neigh_p, rows2d):
    W = 256

    @functools.partial(
        pl.kernel,
        out_type=jax.ShapeDtypeStruct((B, 128), jnp.int32),
        mesh=_SC_MESH(),
    )
    def k(tab_hbm, i_hbm, o_hbm):
        def body(i_vmem, o_vmem):
            pltpu.sync_copy(tab_hbm.at[i_vmem.at[0]], o_vmem)

        pltpu.emit_pipeline(
            body,
            grid=(B // W,),
            in_specs=[pl.BlockSpec((1, W), lambda i: (0, i))],
            out_specs=[pl.BlockSpec((W, 128), lambda i: (i, 0))],
            core_axis_name=("c", "s"),
            dimension_semantics=(pltpu.PARALLEL,),
        )(i_hbm, o_hbm)

    return k(neigh_p, rows2d)


def _sc_gather_xin(xin, tok2d):
    W = 256
    nt = tok2d.shape[1]

    @functools.partial(
        pl.kernel,
        out_type=jax.ShapeDtypeStruct((nt, EMB), jnp.float32),
        mesh=_SC_MESH(),
    )
    def k(tab_hbm, i_hbm, o_hbm):
        def body(i_vmem, o_vmem):
            pltpu.sync_copy(tab_hbm.at[i_vmem.at[0]], o_vmem)

        pltpu.emit_pipeline(
            body,
            grid=(nt // W,),
            in_specs=[pl.BlockSpec((1, W), lambda i: (0, i))],
            out_specs=[pl.BlockSpec((W, EMB), lambda i: (i, 0))],
            core_axis_name=("c", "s"),
            dimension_semantics=(pltpu.PARALLEL,),
        )(i_hbm, o_hbm)

    return k(xin, tok2d)


def _proj_body(feat_ref, lap_ref, wif_ref, wil_ref, o_ref):
    o_ref[...] = (
        jnp.dot(feat_ref[...], wif_ref[...],
                preferred_element_type=jnp.float32)
        + jnp.dot(lap_ref[...], wil_ref[...],
                  preferred_element_type=jnp.float32))


def _proj_kernel(feat, lap, w_in):
    blk = 8192
    nb = (N + 1 + blk - 1) // blk
    return pl.pallas_call(
        _proj_body,
        grid=(nb,),
        in_specs=[
            pl.BlockSpec((blk, D), lambda i: (i, 0)),
            pl.BlockSpec((blk, DL), lambda i: (i, 0)),
            pl.BlockSpec((D, EMB), lambda i: (0, 0)),
            pl.BlockSpec((DL, EMB), lambda i: (0, 0)),
        ],
        out_specs=pl.BlockSpec((blk, EMB), lambda i: (i, 0)),
        out_shape=jax.ShapeDtypeStruct((N + 1, EMB), jnp.float32),
        compiler_params=pltpu.CompilerParams(
            dimension_semantics=("parallel",)),
    )(feat, lap, w_in[:D], w_in[D:])


def _ln(z):
    m = jnp.mean(z, axis=-1, keepdims=True)
    v = jnp.mean(z * z, axis=-1, keepdims=True) - m * m
    return (z - m) / jnp.sqrt(v + 1e-5)


def _softmax(s):
    e = jnp.exp2(s)
    return e * (1.0 / jnp.sum(e, axis=-1, keepdims=True))


def _attn_all(qa, kv, hm, ms, hmo, nq, ng):
    f32 = jnp.float32
    bf16 = jnp.bfloat16

    def mk_scores(g):
        q_grp = lax.slice(qa, (g * nq, 0), ((g + 1) * nq, EMB))
        k_grp = lax.slice(kv, (g * RG, 0), ((g + 1) * RG, EMB))
        qst = jnp.concatenate([q_grp] * H, axis=0) * hm
        return lax.dot_general(qst, k_grp, (((1,), (1,)), ((), ())),
                               preferred_element_type=f32) + ms

    s = {0: mk_scores(0)}
    if ng > 1:
        s[1] = mk_scores(1)
    oparts = []
    for g in range(ng):
        p = _softmax(s.pop(g)).astype(bf16)
        v_grp = lax.slice(kv, (g * RG, EMB), ((g + 1) * RG, 2 * EMB))
        av = jnp.dot(p, v_grp, preferred_element_type=f32)
        o = av[0:nq] * hmo[0:1]
        for h in range(1, H):
            o = o + av[h * nq:(h + 1) * nq] * hmo[h:h + 1]
        oparts.append(o)
        if g + 2 < ng:
            s[g + 2] = mk_scores(g + 2)
    return jnp.concatenate(oparts, axis=0)


def _tc_body(px_ref, ms1_ref, ms2_ref, sel_ref, hm1_ref, hm2_ref, hmo_ref,
             wq_ref, wkv_ref, wo_ref, w1_ref, w2_ref, wd_ref, bd_ref,
             out_ref):
    f32 = jnp.float32
    bf16 = jnp.bfloat16

    x = px_ref[...]
    xb = x.astype(bf16)
    ms1 = ms1_ref[...]
    ms2 = ms2_ref[...]
    hm1 = hm1_ref[...]
    hm2 = hm2_ref[...]
    hmo = hmo_ref[...]

    q = jnp.dot(xb, wq_ref[0], preferred_element_type=f32).astype(bf16)
    kv = jnp.dot(xb, wkv_ref[0], preferred_element_type=f32).astype(bf16)

    o = _attn_all(q, kv, hm1, ms1, hmo, RG, NG)

    x = _ln(x + jnp.dot(o.astype(bf16), wo_ref[0],
                        preferred_element_type=f32))
    xb = x.astype(bf16)
    ff = jnp.dot(jax.nn.relu(jnp.dot(xb, w1_ref[0],
                                     preferred_element_type=f32)).astype(bf16),
                 w2_ref[0], preferred_element_type=f32)
    x = _ln(x + ff)

    xb = x.astype(bf16)
    xs = jnp.dot(sel_ref[...], xb, preferred_element_type=f32)

    q2 = jnp.dot(xs.astype(bf16), wq_ref[1],
                 preferred_element_type=f32).astype(bf16)
    kv2 = jnp.dot(xb, wkv_ref[1], preferred_element_type=f32).astype(bf16)

    o2 = _attn_all(q2, kv2, hm2, ms2, hmo, GS2, NG2)

    xs = _ln(xs + jnp.dot(o2.astype(bf16), wo_ref[1],
                          preferred_element_type=f32))
    xsb = xs.astype(bf16)
    ff2 = jnp.dot(jax.nn.relu(jnp.dot(xsb, w1_ref[1],
                                      preferred_element_type=f32)
                              ).astype(bf16),
                  w2_ref[1], preferred_element_type=f32)
    xs = _ln(xs + ff2)

    out_ref[...] = (jnp.dot(xs.astype(bf16), wd_ref[...],
                            preferred_element_type=f32)
                    + bd_ref[...])


def _tc_transformer(px, wq, wk, wv, wo, w1, w2, wd, bd):
    f32 = jnp.float32
    bf16 = jnp.bfloat16
    bd2 = bd.reshape(1, C)
    wkv = jnp.concatenate([wk, wv], axis=2)
    wq, wkv, wo, w1, w2, wd = (
        w.astype(bf16) for w in (wq, wkv, wo, w1, w2, wd))

    scale = np.log2(np.e) / np.sqrt(DH)

    def band_mask(nrow):
        r = lax.broadcasted_iota(jnp.int32, (H * nrow, RG), 0) % nrow
        c = lax.broadcasted_iota(jnp.int32, (H * nrow, RG), 1)
        if nrow == RG:
            ok = (r // T) == (c // T)
        else:
            ok = (c // T) == r
        return jnp.where(ok, 0.0, -1e30).astype(f32)

    def head_mask(nrow, val):
        hr = lax.broadcasted_iota(jnp.int32, (H * nrow, EMB), 0) // nrow
        lane = lax.broadcasted_iota(jnp.int32, (H * nrow, EMB), 1) // DH
        return jnp.where(hr == lane, val, 0.0).astype(f32)

    ms1 = band_mask(RG)
    ms2 = band_mask(GS2)
    hm1 = head_mask(RG, scale).astype(bf16)
    hm2 = head_mask(GS2, scale).astype(bf16)
    hro = lax.broadcasted_iota(jnp.int32, (H, EMB), 0)
    lno = lax.broadcasted_iota(jnp.int32, (H, EMB), 1) // DH
    hmo = jnp.where(hro == lno, 1.0, 0.0).astype(f32)
    rs = lax.broadcasted_iota(jnp.int32, (BB, R), 0)
    cc = lax.broadcasted_iota(jnp.int32, (BB, R), 1)
    sel = jnp.where(cc == rs * T, 1.0, 0.0).astype(bf16)

    nblk = px.shape[0] // R
    return pl.pallas_call(
        _tc_body,
        grid=(nblk,),
        in_specs=[
            pl.BlockSpec((R, EMB), lambda i: (i, 0)),
            pl.BlockSpec((H * RG, RG), lambda i: (0, 0)),
            pl.BlockSpec((H * GS2, RG), lambda i: (0, 0)),
            pl.BlockSpec((BB, R), lambda i: (0, 0)),
            pl.BlockSpec((H * RG, EMB), lambda i: (0, 0)),
            pl.BlockSpec((H * GS2, EMB), lambda i: (0, 0)),
            pl.BlockSpec((H, EMB), lambda i: (0, 0)),
            pl.BlockSpec((L, EMB, EMB), lambda i: (0, 0, 0)),
            pl.BlockSpec((L, EMB, 2 * EMB), lambda i: (0, 0, 0)),
            pl.BlockSpec((L, EMB, EMB), lambda i: (0, 0, 0)),
            pl.BlockSpec((L, EMB, FF), lambda i: (0, 0, 0)),
            pl.BlockSpec((L, FF, EMB), lambda i: (0, 0, 0)),
            pl.BlockSpec((EMB, C), lambda i: (0, 0)),
            pl.BlockSpec((1, C), lambda i: (0, 0)),
        ],
        out_specs=pl.BlockSpec((BB, C), lambda i: (i, 0)),
        out_shape=jax.ShapeDtypeStruct((nblk * BB, C), f32),
        compiler_params=pltpu.CompilerParams(
            dimension_semantics=("parallel",)),
    )(px, ms1, ms2, sel, hm1, hm2, hmo, wq, wkv, wo, w1, w2, wd, bd2)


def kernel(nodes, neigh, feat, lap, W_in, Wq, Wk, Wv, Wo, W1, W2,
           W_dense, b_dense):
    nodes32 = nodes.astype(jnp.int32)
    neigh32 = neigh.astype(jnp.int32)

    neigh_p = neigh32.reshape(N // 8, 8 * S)
    nbrows = _sc_gather_nbrows(neigh_p, (nodes32 // 8).reshape(1, B))
    j = nodes32[:, None] % 8
    nb = nbrows[:, 0:S]
    for jj in range(1, 8):
        nb = jnp.where(j == jj, nbrows[:, jj * S:(jj + 1) * S], nb)
    tok = jnp.concatenate([nodes32[:, None], nb], axis=1)

    xin = _proj_kernel(feat, lap, W_in)

    NCHUNK = 4
    BC = B // NCHUNK
    outs = []
    for c in range(NCHUNK):
        tok_c = lax.slice(tok, (c * BC, 0), ((c + 1) * BC, T))
        px_c = _sc_gather_xin(xin, tok_c.reshape(1, BC * T))
        outs.append(_tc_transformer(px_c, Wq, Wk, Wv, Wo, W1, W2,
                                    W_dense, b_dense))
    return jnp.concatenate(outs, axis=0)


# speedup vs baseline: 3.4316x; 1.0381x over previous
import functools

import jax
import jax.numpy as jnp
import numpy as np
from jax import lax
from jax.experimental import pallas as pl
from jax.experimental.pallas import tpu as pltpu
from jax.experimental.pallas import tpu_sc as plsc

N = 100000
D = 128
DL = 16
S = 16
B = 8192
EMB = 128
H = 4
L = 2
FF = 256
C = 40
T = S + 1
BT = B * T
DH = EMB // H

BB = 128
R = BB * T
GS = 8
RG = GS * T
NG = BB // GS
GS2 = 8
RG2 = GS2 * T
NG2 = BB // GS2
NBLK = B // BB

_SC_MESH = functools.partial(
    plsc.VectorSubcoreMesh, core_axis_name="c", subcore_axis_name="s"
)


def _sc_gather_nbrows(neigh_p, rows2d):
    W = 256

    @functools.partial(
        pl.kernel,
        out_type=jax.ShapeDtypeStruct((B, 128), jnp.int32),
        mesh=_SC_MESH(),
    )
    def k(tab_hbm, i_hbm, o_hbm):
        def body(i_vmem, o_vmem):
            pltpu.sync_copy(tab_hbm.at[i_vmem.at[0]], o_vmem)

        pltpu.emit_pipeline(
            body,
            grid=(B // W,),
            in_specs=[pl.BlockSpec((1, W), lambda i: (0, i))],
            out_specs=[pl.BlockSpec((W, 128), lambda i: (i, 0))],
            core_axis_name=("c", "s"),
            dimension_semantics=(pltpu.PARALLEL,),
        )(i_hbm, o_hbm)

    return k(neigh_p, rows2d)


def _sc_gather_xin(xin, tok2d):
    W = 256
    nt = tok2d.shape[1]

    @functools.partial(
        pl.kernel,
        out_type=jax.ShapeDtypeStruct((nt, EMB), jnp.float32),
        mesh=_SC_MESH(),
    )
    def k(tab_hbm, i_hbm, o_hbm):
        def body(i_vmem, o_vmem):
            pltpu.sync_copy(tab_hbm.at[i_vmem.at[0]], o_vmem)

        pltpu.emit_pipeline(
            body,
            grid=(nt // W,),
            in_specs=[pl.BlockSpec((1, W), lambda i: (0, i))],
            out_specs=[pl.BlockSpec((W, EMB), lambda i: (i, 0))],
            core_axis_name=("c", "s"),
            dimension_semantics=(pltpu.PARALLEL,),
        )(i_hbm, o_hbm)

    return k(xin, tok2d)


def _proj_body(feat_ref, lap_ref, wif_ref, wil_ref, o_ref):
    o_ref[...] = (
        jnp.dot(feat_ref[...], wif_ref[...],
                preferred_element_type=jnp.float32)
        + jnp.dot(lap_ref[...], wil_ref[...],
                  preferred_element_type=jnp.float32))


def _proj_kernel(feat, lap, w_in):
    blk = 8192
    nb = (N + 1 + blk - 1) // blk
    return pl.pallas_call(
        _proj_body,
        grid=(nb,),
        in_specs=[
            pl.BlockSpec((blk, D), lambda i: (i, 0)),
            pl.BlockSpec((blk, DL), lambda i: (i, 0)),
            pl.BlockSpec((D, EMB), lambda i: (0, 0)),
            pl.BlockSpec((DL, EMB), lambda i: (0, 0)),
        ],
        out_specs=pl.BlockSpec((blk, EMB), lambda i: (i, 0)),
        out_shape=jax.ShapeDtypeStruct((N + 1, EMB), jnp.float32),
        compiler_params=pltpu.CompilerParams(
            dimension_semantics=("parallel",)),
    )(feat, lap, w_in[:D], w_in[D:])


def _ln(z):
    m = jnp.mean(z, axis=-1, keepdims=True)
    v = jnp.mean(z * z, axis=-1, keepdims=True) - m * m
    return (z - m) / jnp.sqrt(v + 1e-5)


def _softmax(s):
    e = jnp.exp2(s)
    return e * (1.0 / jnp.sum(e, axis=-1, keepdims=True))


def _attn_all(qa, kv, hm, ms, hmo, nq, ng):
    f32 = jnp.float32
    bf16 = jnp.bfloat16

    def mk_scores(g):
        q_grp = lax.slice(qa, (g * nq, 0), ((g + 1) * nq, EMB))
        k_grp = lax.slice(kv, (g * RG, 0), ((g + 1) * RG, EMB))
        qst = jnp.concatenate([q_grp] * H, axis=0) * hm
        return lax.dot_general(qst, k_grp, (((1,), (1,)), ((), ())),
                               preferred_element_type=f32) + ms

    DEPTH = 3
    s = {g: mk_scores(g) for g in range(min(DEPTH, ng))}
    oparts = []
    for g in range(ng):
        p = _softmax(s.pop(g)).astype(bf16)
        v_grp = lax.slice(kv, (g * RG, EMB), ((g + 1) * RG, 2 * EMB))
        av = jnp.dot(p, v_grp, preferred_element_type=f32)
        o = av[0:nq] * hmo[0:1]
        for h in range(1, H):
            o = o + av[h * nq:(h + 1) * nq] * hmo[h:h + 1]
        oparts.append(o)
        if g + DEPTH < ng:
            s[g + DEPTH] = mk_scores(g + DEPTH)
    return jnp.concatenate(oparts, axis=0)


def _tc_body(px_ref, ms1_ref, ms2_ref, hm1_ref, hm2_ref, hmo_ref,
             wq_ref, wkv_ref, wo_ref, w1_ref, w2_ref, wd_ref, bd_ref,
             out_ref):
    f32 = jnp.float32
    bf16 = jnp.bfloat16

    x = px_ref[...]
    xb = x.astype(bf16)
    ms1 = ms1_ref[...]
    ms2 = ms2_ref[...]
    hm1 = hm1_ref[...]
    hm2 = hm2_ref[...]
    hmo = hmo_ref[...]

    q = jnp.dot(xb, wq_ref[0], preferred_element_type=f32).astype(bf16)
    kv = jnp.dot(xb, wkv_ref[0], preferred_element_type=f32).astype(bf16)

    o = _attn_all(q, kv, hm1, ms1, hmo, RG, NG)

    x = _ln(x + jnp.dot(o.astype(bf16), wo_ref[0],
                        preferred_element_type=f32))
    xb = x.astype(bf16)
    ff = jnp.dot(jax.nn.relu(jnp.dot(xb, w1_ref[0],
                                     preferred_element_type=f32)).astype(bf16),
                 w2_ref[0], preferred_element_type=f32)
    x = _ln(x + ff)

    xb = x.astype(bf16)
    xs = jnp.reshape(x, (BB, T, EMB))[:, 0, :]

    q2 = jnp.dot(xs.astype(bf16), wq_ref[1],
                 preferred_element_type=f32).astype(bf16)
    kv2 = jnp.dot(xb, wkv_ref[1], preferred_element_type=f32).astype(bf16)

    o2 = _attn_all(q2, kv2, hm2, ms2, hmo, GS2, NG2)

    xs = _ln(xs + jnp.dot(o2.astype(bf16), wo_ref[1],
                          preferred_element_type=f32))
    xsb = xs.astype(bf16)
    ff2 = jnp.dot(jax.nn.relu(jnp.dot(xsb, w1_ref[1],
                                      preferred_element_type=f32)
                              ).astype(bf16),
                  w2_ref[1], preferred_element_type=f32)
    xs = _ln(xs + ff2)

    out_ref[...] = (jnp.dot(xs.astype(bf16), wd_ref[...],
                            preferred_element_type=f32)
                    + bd_ref[...])


def _tc_transformer(px, wq, wk, wv, wo, w1, w2, wd, bd):
    f32 = jnp.float32
    bf16 = jnp.bfloat16
    bd2 = bd.reshape(1, C)
    wkv = jnp.concatenate([wk, wv], axis=2)
    wq, wkv, wo, w1, w2, wd = (
        w.astype(bf16) for w in (wq, wkv, wo, w1, w2, wd))

    scale = np.log2(np.e) / np.sqrt(DH)

    def band_mask(nrow):
        r = lax.broadcasted_iota(jnp.int32, (H * nrow, RG), 0) % nrow
        c = lax.broadcasted_iota(jnp.int32, (H * nrow, RG), 1)
        if nrow == RG:
            ok = (r // T) == (c // T)
        else:
            ok = (c // T) == r
        return jnp.where(ok, 0.0, -1e30).astype(f32)

    def head_mask(nrow, val):
        hr = lax.broadcasted_iota(jnp.int32, (H * nrow, EMB), 0) // nrow
        lane = lax.broadcasted_iota(jnp.int32, (H * nrow, EMB), 1) // DH
        return jnp.where(hr == lane, val, 0.0).astype(f32)

    ms1 = band_mask(RG)
    ms2 = band_mask(GS2)
    hm1 = head_mask(RG, scale).astype(bf16)
    hm2 = head_mask(GS2, scale).astype(bf16)
    hro = lax.broadcasted_iota(jnp.int32, (H, EMB), 0)
    lno = lax.broadcasted_iota(jnp.int32, (H, EMB), 1) // DH
    hmo = jnp.where(hro == lno, 1.0, 0.0).astype(f32)
    nblk = px.shape[0] // R
    return pl.pallas_call(
        _tc_body,
        grid=(nblk,),
        in_specs=[
            pl.BlockSpec((R, EMB), lambda i: (i, 0)),
            pl.BlockSpec((H * RG, RG), lambda i: (0, 0)),
            pl.BlockSpec((H * GS2, RG), lambda i: (0, 0)),
            pl.BlockSpec((H * RG, EMB), lambda i: (0, 0)),
            pl.BlockSpec((H * GS2, EMB), lambda i: (0, 0)),
            pl.BlockSpec((H, EMB), lambda i: (0, 0)),
            pl.BlockSpec((L, EMB, EMB), lambda i: (0, 0, 0)),
            pl.BlockSpec((L, EMB, 2 * EMB), lambda i: (0, 0, 0)),
            pl.BlockSpec((L, EMB, EMB), lambda i: (0, 0, 0)),
            pl.BlockSpec((L, EMB, FF), lambda i: (0, 0, 0)),
            pl.BlockSpec((L, FF, EMB), lambda i: (0, 0, 0)),
            pl.BlockSpec((EMB, C), lambda i: (0, 0)),
            pl.BlockSpec((1, C), lambda i: (0, 0)),
        ],
        out_specs=pl.BlockSpec((BB, C), lambda i: (i, 0)),
        out_shape=jax.ShapeDtypeStruct((nblk * BB, C), f32),
        compiler_params=pltpu.CompilerParams(
            dimension_semantics=("parallel",)),
    )(px, ms1, ms2, hm1, hm2, hmo, wq, wkv, wo, w1, w2, wd, bd2)


def kernel(nodes, neigh, feat, lap, W_in, Wq, Wk, Wv, Wo, W1, W2,
           W_dense, b_dense):
    nodes32 = nodes.astype(jnp.int32)
    neigh32 = neigh.astype(jnp.int32)

    neigh_p = neigh32.reshape(N // 8, 8 * S)
    nbrows = _sc_gather_nbrows(neigh_p, (nodes32 // 8).reshape(1, B))
    j = nodes32[:, None] % 8
    nb = nbrows[:, 0:S]
    for jj in range(1, 8):
        nb = jnp.where(j == jj, nbrows[:, jj * S:(jj + 1) * S], nb)
    tok = jnp.concatenate([nodes32[:, None], nb], axis=1)

    xin = _proj_kernel(feat, lap, W_in)

    NCHUNK = 4
    BC = B // NCHUNK
    outs = []
    for c in range(NCHUNK):
        tok_c = lax.slice(tok, (c * BC, 0), ((c + 1) * BC, T))
        px_c = _sc_gather_xin(xin, tok_c.reshape(1, BC * T))
        outs.append(_tc_transformer(px_c, Wq, Wk, Wv, Wo, W1, W2,
                                    W_dense, b_dense))
    return jnp.concatenate(outs, axis=0)


# speedup vs baseline: 3.6535x; 1.0647x over previous
import functools

import jax
import jax.numpy as jnp
import numpy as np
from jax import lax
from jax.experimental import pallas as pl
from jax.experimental.pallas import tpu as pltpu
from jax.experimental.pallas import tpu_sc as plsc

N = 100000
D = 128
DL = 16
S = 16
B = 8192
EMB = 128
H = 4
L = 2
FF = 256
C = 40
T = S + 1
BT = B * T
DH = EMB // H

BB = 256
R = BB * T
GS = 8
RG = GS * T
NG = BB // GS
GS2 = 8
RG2 = GS2 * T
NG2 = BB // GS2
NBLK = B // BB

_SC_MESH = functools.partial(
    plsc.VectorSubcoreMesh, core_axis_name="c", subcore_axis_name="s"
)


def _sc_gather_nbrows(neigh_p, rows2d):
    W = 256

    @functools.partial(
        pl.kernel,
        out_type=jax.ShapeDtypeStruct((B, 128), jnp.int32),
        mesh=_SC_MESH(),
    )
    def k(tab_hbm, i_hbm, o_hbm):
        def body(i_vmem, o_vmem):
            pltpu.sync_copy(tab_hbm.at[i_vmem.at[0]], o_vmem)

        pltpu.emit_pipeline(
            body,
            grid=(B // W,),
            in_specs=[pl.BlockSpec((1, W), lambda i: (0, i))],
            out_specs=[pl.BlockSpec((W, 128), lambda i: (i, 0))],
            core_axis_name=("c", "s"),
            dimension_semantics=(pltpu.PARALLEL,),
        )(i_hbm, o_hbm)

    return k(neigh_p, rows2d)


def _sc_gather_xin(xin, tok2d):
    W = 256
    nt = tok2d.shape[1]

    @functools.partial(
        pl.kernel,
        out_type=jax.ShapeDtypeStruct((nt, EMB), jnp.float32),
        mesh=_SC_MESH(),
    )
    def k(tab_hbm, i_hbm, o_hbm):
        def body(i_vmem, o_vmem):
            pltpu.sync_copy(tab_hbm.at[i_vmem.at[0]], o_vmem)

        pltpu.emit_pipeline(
            body,
            grid=(nt // W,),
            in_specs=[pl.BlockSpec((1, W), lambda i: (0, i))],
            out_specs=[pl.BlockSpec((W, EMB), lambda i: (i, 0))],
            core_axis_name=("c", "s"),
            dimension_semantics=(pltpu.PARALLEL,),
        )(i_hbm, o_hbm)

    return k(xin, tok2d)


def _proj_body(feat_ref, lap_ref, wif_ref, wil_ref, o_ref):
    bf16 = jnp.bfloat16
    o_ref[...] = (
        jnp.dot(feat_ref[...].astype(bf16), wif_ref[...],
                preferred_element_type=jnp.float32)
        + jnp.dot(lap_ref[...].astype(bf16), wil_ref[...],
                  preferred_element_type=jnp.float32))


def _proj_kernel(feat, lap, w_in):
    blk = 8192
    nb = (N + 1 + blk - 1) // blk
    return pl.pallas_call(
        _proj_body,
        grid=(nb,),
        in_specs=[
            pl.BlockSpec((blk, D), lambda i: (i, 0)),
            pl.BlockSpec((blk, DL), lambda i: (i, 0)),
            pl.BlockSpec((D, EMB), lambda i: (0, 0)),
            pl.BlockSpec((DL, EMB), lambda i: (0, 0)),
        ],
        out_specs=pl.BlockSpec((blk, EMB), lambda i: (i, 0)),
        out_shape=jax.ShapeDtypeStruct((N + 1, EMB), jnp.float32),
        compiler_params=pltpu.CompilerParams(
            dimension_semantics=("parallel",)),
    )(feat, lap, w_in[:D].astype(jnp.bfloat16), w_in[D:].astype(jnp.bfloat16))


def _ln(z):
    m = jnp.mean(z, axis=-1, keepdims=True)
    v = jnp.mean(z * z, axis=-1, keepdims=True) - m * m
    return (z - m) / jnp.sqrt(v + 1e-5)


def _softmax(s):
    e = jnp.exp2(s)
    return e * (1.0 / jnp.sum(e, axis=-1, keepdims=True))


def _attn_all(qa, kv, hm, ms, hmo, nq, ng):
    f32 = jnp.float32
    bf16 = jnp.bfloat16

    def mk_scores(g):
        q_grp = lax.slice(qa, (g * nq, 0), ((g + 1) * nq, EMB))
        k_grp = lax.slice(kv, (g * RG, 0), ((g + 1) * RG, EMB))
        qst = jnp.concatenate([q_grp] * H, axis=0) * hm
        return lax.dot_general(qst, k_grp, (((1,), (1,)), ((), ())),
                               preferred_element_type=f32) + ms

    DEPTH = 3
    s = {g: mk_scores(g) for g in range(min(DEPTH, ng))}
    oparts = []
    for g in range(ng):
        p = _softmax(s.pop(g)).astype(bf16)
        v_grp = lax.slice(kv, (g * RG, EMB), ((g + 1) * RG, 2 * EMB))
        av = jnp.dot(p, v_grp, preferred_element_type=f32)
        o = av[0:nq] * hmo[0:1]
        for h in range(1, H):
            o = o + av[h * nq:(h + 1) * nq] * hmo[h:h + 1]
        oparts.append(o)
        if g + DEPTH < ng:
            s[g + DEPTH] = mk_scores(g + DEPTH)
    return jnp.concatenate(oparts, axis=0)


def _tc_body(px_ref, ms1_ref, ms2_ref, hm1_ref, hm2_ref, hmo_ref,
             wq_ref, wkv_ref, wo_ref, w1_ref, w2_ref, wd_ref, bd_ref,
             out_ref):
    f32 = jnp.float32
    bf16 = jnp.bfloat16

    x = px_ref[...]
    xb = x.astype(bf16)
    ms1 = ms1_ref[...]
    ms2 = ms2_ref[...]
    hm1 = hm1_ref[...]
    hm2 = hm2_ref[...]
    hmo = hmo_ref[...]

    q = jnp.dot(xb, wq_ref[0], preferred_element_type=f32).astype(bf16)
    kv = jnp.dot(xb, wkv_ref[0], preferred_element_type=f32).astype(bf16)

    o = _attn_all(q, kv, hm1, ms1, hmo, RG, NG)

    x = _ln(x + jnp.dot(o.astype(bf16), wo_ref[0],
                        preferred_element_type=f32))
    xb = x.astype(bf16)
    ff = jnp.dot(jax.nn.relu(jnp.dot(xb, w1_ref[0],
                                     preferred_element_type=f32)).astype(bf16),
                 w2_ref[0], preferred_element_type=f32)
    x = _ln(x + ff)

    xb = x.astype(bf16)
    xs = jnp.reshape(x, (BB, T, EMB))[:, 0, :]

    q2 = jnp.dot(xs.astype(bf16), wq_ref[1],
                 preferred_element_type=f32).astype(bf16)
    kv2 = jnp.dot(xb, wkv_ref[1], preferred_element_type=f32).astype(bf16)

    o2 = _attn_all(q2, kv2, hm2, ms2, hmo, GS2, NG2)

    xs = _ln(xs + jnp.dot(o2.astype(bf16), wo_ref[1],
                          preferred_element_type=f32))
    xsb = xs.astype(bf16)
    ff2 = jnp.dot(jax.nn.relu(jnp.dot(xsb, w1_ref[1],
                                      preferred_element_type=f32)
                              ).astype(bf16),
                  w2_ref[1], preferred_element_type=f32)
    xs = _ln(xs + ff2)

    out_ref[...] = (jnp.dot(xs.astype(bf16), wd_ref[...],
                            preferred_element_type=f32)
                    + bd_ref[...])


def _tc_transformer(px, wq, wk, wv, wo, w1, w2, wd, bd):
    f32 = jnp.float32
    bf16 = jnp.bfloat16
    bd2 = bd.reshape(1, C)
    wkv = jnp.concatenate([wk, wv], axis=2)
    wq, wkv, wo, w1, w2, wd = (
        w.astype(bf16) for w in (wq, wkv, wo, w1, w2, wd))

    scale = np.log2(np.e) / np.sqrt(DH)

    def band_mask(nrow):
        r = lax.broadcasted_iota(jnp.int32, (H * nrow, RG), 0) % nrow
        c = lax.broadcasted_iota(jnp.int32, (H * nrow, RG), 1)
        if nrow == RG:
            ok = (r // T) == (c // T)
        else:
            ok = (c // T) == r
        return jnp.where(ok, 0.0, -1e30).astype(f32)

    def head_mask(nrow, val):
        hr = lax.broadcasted_iota(jnp.int32, (H * nrow, EMB), 0) // nrow
        lane = lax.broadcasted_iota(jnp.int32, (H * nrow, EMB), 1) // DH
        return jnp.where(hr == lane, val, 0.0).astype(f32)

    ms1 = band_mask(RG)
    ms2 = band_mask(GS2)
    hm1 = head_mask(RG, scale).astype(bf16)
    hm2 = head_mask(GS2, scale).astype(bf16)
    hro = lax.broadcasted_iota(jnp.int32, (H, EMB), 0)
    lno = lax.broadcasted_iota(jnp.int32, (H, EMB), 1) // DH
    hmo = jnp.where(hro == lno, 1.0, 0.0).astype(f32)
    nblk = px.shape[0] // R
    return pl.pallas_call(
        _tc_body,
        grid=(nblk,),
        in_specs=[
            pl.BlockSpec((R, EMB), lambda i: (i, 0)),
            pl.BlockSpec((H * RG, RG), lambda i: (0, 0)),
            pl.BlockSpec((H * GS2, RG), lambda i: (0, 0)),
            pl.BlockSpec((H * RG, EMB), lambda i: (0, 0)),
            pl.BlockSpec((H * GS2, EMB), lambda i: (0, 0)),
            pl.BlockSpec((H, EMB), lambda i: (0, 0)),
            pl.BlockSpec((L, EMB, EMB), lambda i: (0, 0, 0)),
            pl.BlockSpec((L, EMB, 2 * EMB), lambda i: (0, 0, 0)),
            pl.BlockSpec((L, EMB, EMB), lambda i: (0, 0, 0)),
            pl.BlockSpec((L, EMB, FF), lambda i: (0, 0, 0)),
            pl.BlockSpec((L, FF, EMB), lambda i: (0, 0, 0)),
            pl.BlockSpec((EMB, C), lambda i: (0, 0)),
            pl.BlockSpec((1, C), lambda i: (0, 0)),
        ],
        out_specs=pl.BlockSpec((BB, C), lambda i: (i, 0)),
        out_shape=jax.ShapeDtypeStruct((nblk * BB, C), f32),
        compiler_params=pltpu.CompilerParams(
            dimension_semantics=("parallel",)),
    )(px, ms1, ms2, hm1, hm2, hmo, wq, wkv, wo, w1, w2, wd, bd2)


def kernel(nodes, neigh, feat, lap, W_in, Wq, Wk, Wv, Wo, W1, W2,
           W_dense, b_dense):
    nodes32 = nodes.astype(jnp.int32)
    neigh32 = neigh.astype(jnp.int32)

    neigh_p = neigh32.reshape(N // 8, 8 * S)
    nbrows = _sc_gather_nbrows(neigh_p, (nodes32 // 8).reshape(1, B))
    j = nodes32[:, None] % 8
    nb = nbrows[:, 0:S]
    for jj in range(1, 8):
        nb = jnp.where(j == jj, nbrows[:, jj * S:(jj + 1) * S], nb)
    tok = jnp.concatenate([nodes32[:, None], nb], axis=1)

    xin = _proj_kernel(feat, lap, W_in)

    NCHUNK = 4
    BC = B // NCHUNK
    outs = []
    for c in range(NCHUNK):
        tok_c = lax.slice(tok, (c * BC, 0), ((c + 1) * BC, T))
        px_c = _sc_gather_xin(xin, tok_c.reshape(1, BC * T))
        outs.append(_tc_transformer(px_c, Wq, Wk, Wv, Wo, W1, W2,
                                    W_dense, b_dense))
    return jnp.concatenate(outs, axis=0)
